# baseline (device time: 1200323 ns/iter reference)
import jax
import jax.numpy as jnp
from jax import lax
from jax.experimental import pallas as pl
from jax.experimental.pallas import tpu as pltpu

N_DEV = 32
B, SQ, SKV, D_MODEL = 2, 512, 512, 768
HQ_PER = 8
DH = 64
H_PER = HQ_PER * DH
BLK = 64


def kernel(x, Wq, K_ext, V_ext, Wo):
    my = lax.axis_index("i")
    K_loc = lax.dynamic_slice_in_dim(K_ext, my * HQ_PER, HQ_PER, axis=2)
    V_loc = lax.dynamic_slice_in_dim(V_ext, my * HQ_PER, HQ_PER, axis=2)
    kb = jnp.transpose(K_loc, (0, 2, 1, 3)).astype(jnp.bfloat16)
    vb = jnp.transpose(V_loc, (0, 2, 1, 3)).astype(jnp.bfloat16)
    xb = x.astype(jnp.bfloat16)
    wqb = Wq.astype(jnp.bfloat16)
    wob = Wo.astype(jnp.bfloat16)

    def body(x_ref, wq_ref, k_ref, v_ref, wo_ref, out_ref,
             q_ref, ctx_ref, comm_ref, send_sems, recv_sems, credit_sems):
        my_pos = lax.axis_index("i")
        left = lax.rem(my_pos + N_DEV - 1, N_DEV)
        right = lax.rem(my_pos + 1, N_DEV)

        barrier_sem = pltpu.get_barrier_semaphore()
        for nbr in (left, right):
            pl.semaphore_signal(
                barrier_sem, inc=1,
                device_id=(nbr,), device_id_type=pl.DeviceIdType.MESH,
            )
        pl.semaphore_wait(barrier_sem, 2)

        for b in range(B):
            q_ref[b] = jnp.dot(
                x_ref[b], wq_ref[...], preferred_element_type=jnp.float32
            ).astype(jnp.bfloat16)

        qblk = lax.broadcasted_iota(jnp.int32, (SQ, SKV), 0) // BLK
        kblk = lax.broadcasted_iota(jnp.int32, (SQ, SKV), 1) // BLK
        mask = kblk <= qblk

        for b in range(B):
            for h in range(HQ_PER):
                qh = q_ref[b, :, h * DH:(h + 1) * DH]
                s = lax.dot_general(
                    qh, k_ref[b, h],
                    (((1,), (1,)), ((), ())),
                    preferred_element_type=jnp.float32,
                ) * 0.125
                s = jnp.where(mask, s, -1e9)
                m = jnp.max(s, axis=1, keepdims=True)
                w = jnp.exp(s - m)
                w = w / jnp.sum(w, axis=1, keepdims=True)
                ctx = jnp.dot(
                    w.astype(jnp.bfloat16), v_ref[b, h],
                    preferred_element_type=jnp.float32,
                )
                ctx_ref[b, :, h * DH:(h + 1) * DH] = ctx.astype(jnp.bfloat16)

        for b in range(B):
            out_ref[b] = jnp.dot(
                ctx_ref[b], wo_ref[...], preferred_element_type=jnp.float32
            )
        comm_ref[0] = out_ref[...]

        for h in range(N_DEV - 1):
            s_slot = h % 2
            r_slot = (h + 1) % 2
            if h >= 2:
                pl.semaphore_wait(credit_sems.at[r_slot], 1)
            rdma = pltpu.make_async_remote_copy(
                src_ref=comm_ref.at[s_slot],
                dst_ref=comm_ref.at[r_slot],
                send_sem=send_sems.at[s_slot],
                recv_sem=recv_sems.at[r_slot],
                device_id=(right,),
                device_id_type=pl.DeviceIdType.MESH,
            )
            rdma.start()
            rdma.wait()
            out_ref[...] += comm_ref[r_slot]
            if h < N_DEV - 3:
                pl.semaphore_signal(
                    credit_sems.at[r_slot], inc=1,
                    device_id=(left,), device_id_type=pl.DeviceIdType.MESH,
                )

    return pl.pallas_call(
        body,
        out_shape=jax.ShapeDtypeStruct((B, SQ, D_MODEL), jnp.float32),
        in_specs=[pl.BlockSpec(memory_space=pltpu.VMEM)] * 5,
        out_specs=pl.BlockSpec(memory_space=pltpu.VMEM),
        scratch_shapes=[
            pltpu.VMEM((B, SQ, H_PER), jnp.bfloat16),
            pltpu.VMEM((B, SQ, H_PER), jnp.bfloat16),
            pltpu.VMEM((2, B, SQ, D_MODEL), jnp.float32),
            pltpu.SemaphoreType.DMA((2,)),
            pltpu.SemaphoreType.DMA((2,)),
            pltpu.SemaphoreType.REGULAR((2,)),
        ],
        compiler_params=pltpu.CompilerParams(collective_id=0),
    )(xb, wqb, kb, vb, wob)


# device time: 180229 ns/iter; 6.6600x vs baseline; 6.6600x over previous
import jax
import jax.numpy as jnp
from jax import lax
from jax.experimental import pallas as pl
from jax.experimental.pallas import tpu as pltpu

N_DEV = 32
LOG_DEV = 5
B, SQ, SKV, D_MODEL = 2, 512, 512, 768
HQ_PER = 8
DH = 64
H_PER = HQ_PER * DH
BLK = 64
ROWS = B * SQ

_SOFF = [0, 512, 768, 896, 960]


def kernel(x, Wq, K_ext, V_ext, Wo):
    my = lax.axis_index("i")
    K_loc = lax.dynamic_slice_in_dim(K_ext, my * HQ_PER, HQ_PER, axis=2)
    V_loc = lax.dynamic_slice_in_dim(V_ext, my * HQ_PER, HQ_PER, axis=2)
    kb = jnp.transpose(K_loc, (0, 2, 1, 3)).astype(jnp.bfloat16)
    vb = jnp.transpose(V_loc, (0, 2, 1, 3)).astype(jnp.bfloat16)
    xb = x.astype(jnp.bfloat16)
    wqb = Wq.astype(jnp.bfloat16)
    wob = Wo.astype(jnp.bfloat16)

    def body(x_ref, wq_ref, k_ref, v_ref, wo_ref, out_ref,
             q_ref, ctx_ref, acc_ref, stage_ref,
             rs_send_sems, rs_recv_sems, ag_send_sems, ag_recv_sems):
        my_pos = lax.axis_index("i")

        barrier_sem = pltpu.get_barrier_semaphore()
        for k in range(LOG_DEV):
            partner = lax.bitwise_xor(my_pos, 1 << k)
            pl.semaphore_signal(
                barrier_sem, inc=1,
                device_id=(partner,), device_id_type=pl.DeviceIdType.MESH,
            )
        pl.semaphore_wait(barrier_sem, LOG_DEV)

        for b in range(B):
            q_ref[b] = jnp.dot(
                x_ref[b], wq_ref[...], preferred_element_type=jnp.float32
            ).astype(jnp.bfloat16)

        qblk = lax.broadcasted_iota(jnp.int32, (SQ, SKV), 0) // BLK
        kblk = lax.broadcasted_iota(jnp.int32, (SQ, SKV), 1) // BLK
        mask = kblk <= qblk

        for b in range(B):
            for h in range(HQ_PER):
                qh = q_ref[b, :, h * DH:(h + 1) * DH]
                s = lax.dot_general(
                    qh, k_ref[b, h],
                    (((1,), (1,)), ((), ())),
                    preferred_element_type=jnp.float32,
                ) * 0.125
                s = jnp.where(mask, s, -1e9)
                m = jnp.max(s, axis=1, keepdims=True)
                w = jnp.exp(s - m)
                w = w / jnp.sum(w, axis=1, keepdims=True)
                ctx = jnp.dot(
                    w.astype(jnp.bfloat16), v_ref[b, h],
                    preferred_element_type=jnp.float32,
                )
                ctx_ref[b, :, h * DH:(h + 1) * DH] = ctx.astype(jnp.bfloat16)

        for b in range(B):
            acc_ref[b * SQ:(b + 1) * SQ, :] = jnp.dot(
                ctx_ref[b], wo_ref[...], preferred_element_type=jnp.float32
            )

        start = my_pos * 0
        for k in range(LOG_DEV):
            size = ROWS >> k
            half = size >> 1
            partner = lax.bitwise_xor(my_pos, 1 << k)
            bitk = lax.bitwise_and(lax.shift_right_logical(my_pos, k), 1)
            keep_start = start + bitk * half
            send_start = start + (1 - bitk) * half
            rdma = pltpu.make_async_remote_copy(
                src_ref=acc_ref.at[pl.ds(send_start, half)],
                dst_ref=stage_ref.at[pl.ds(_SOFF[k], half)],
                send_sem=rs_send_sems.at[k],
                recv_sem=rs_recv_sems.at[k],
                device_id=(partner,),
                device_id_type=pl.DeviceIdType.MESH,
            )
            rdma.start()
            rdma.wait()
            acc_ref[pl.ds(keep_start, half), :] = (
                acc_ref[pl.ds(keep_start, half), :]
                + stage_ref[pl.ds(_SOFF[k], half), :]
            )
            start = keep_start

        for k in reversed(range(LOG_DEV)):
            size = ROWS >> (k + 1)
            partner = lax.bitwise_xor(my_pos, 1 << k)
            bitk = lax.bitwise_and(lax.shift_right_logical(my_pos, k), 1)
            rdma = pltpu.make_async_remote_copy(
                src_ref=acc_ref.at[pl.ds(start, size)],
                dst_ref=acc_ref.at[pl.ds(start, size)],
                send_sem=ag_send_sems.at[k],
                recv_sem=ag_recv_sems.at[k],
                device_id=(partner,),
                device_id_type=pl.DeviceIdType.MESH,
            )
            rdma.start()
            rdma.wait()
            start = start - bitk * size

        for b in range(B):
            out_ref[b] = acc_ref[b * SQ:(b + 1) * SQ, :]

    return pl.pallas_call(
        body,
        out_shape=jax.ShapeDtypeStruct((B, SQ, D_MODEL), jnp.float32),
        in_specs=[pl.BlockSpec(memory_space=pltpu.VMEM)] * 5,
        out_specs=pl.BlockSpec(memory_space=pltpu.VMEM),
        scratch_shapes=[
            pltpu.VMEM((B, SQ, H_PER), jnp.bfloat16),
            pltpu.VMEM((B, SQ, H_PER), jnp.bfloat16),
            pltpu.VMEM((ROWS, D_MODEL), jnp.float32),
            pltpu.VMEM((992, D_MODEL), jnp.float32),
            pltpu.SemaphoreType.DMA((LOG_DEV,)),
            pltpu.SemaphoreType.DMA((LOG_DEV,)),
            pltpu.SemaphoreType.DMA((LOG_DEV,)),
            pltpu.SemaphoreType.DMA((LOG_DEV,)),
        ],
        compiler_params=pltpu.CompilerParams(collective_id=0),
    )(xb, wqb, kb, vb, wob)
